# baseline (speedup 1.0000x reference)
import functools

import jax
import jax.numpy as jnp
from jax import lax
from jax.experimental import pallas as pl
from jax.experimental.pallas import tpu as pltpu
from jax.experimental.pallas import tpu_sc as plsc

_B = 16384
_NF = 26
_NC = 13
_V = 100000
_D = 16
_WD = 128
_H1 = 256
_H2 = 128
_NCLS = 2
_DE = _NF * _D

_NCORES = 2
_NSUB = 16
_NW = _NCORES * _NSUB
_PPW = _DE // _NW
_BCH = 8192
_NBCH = _B // _BCH


_UNR = 16


def _sc_plane_gather_body(t2_hbm, idxT_hbm, out_hbm, plane_v, iv_v, ob_v):
    wid = lax.axis_index("s") * _NCORES + lax.axis_index("c")

    def plane_step(p, prev_f):
        plane = wid * _PPW + p
        f = plane // _D
        d = plane % _D

        @pl.when(f != prev_f)
        def _():
            pltpu.sync_copy(idxT_hbm.at[f, :], iv_v)

        pltpu.sync_copy(t2_hbm.at[f, d, :], plane_v)

        def bchunk(c, carry2):
            def step(i, carry3):
                base = i * (16 * _UNR)
                for j in range(_UNR):
                    o = base + j * 16
                    ivv = iv_v[pl.ds(c * _BCH + o, 16)]
                    ob_v[pl.ds(o, 16)] = plsc.load_gather(plane_v, [ivv])
                return carry3

            lax.fori_loop(0, _BCH // (16 * _UNR), step, 0)
            pltpu.sync_copy(ob_v, out_hbm.at[plane, pl.ds(c * _BCH, _BCH)])
            return carry2

        lax.fori_loop(0, _NBCH, bchunk, 0)
        return f

    lax.fori_loop(0, _PPW, plane_step, -1)


@functools.cache
def _sc_gather():
    return pl.kernel(
        _sc_plane_gather_body,
        out_type=jax.ShapeDtypeStruct((_DE, _B), jnp.float32),
        mesh=plsc.VectorSubcoreMesh(core_axis_name="c", subcore_axis_name="s",
                                    num_cores=_NCORES, num_subcores=_NSUB),
        scratch_types=[
            pltpu.VMEM((_V,), jnp.float32),
            pltpu.VMEM((_B,), jnp.int32),
            pltpu.VMEM((_BCH,), jnp.float32),
        ],
        compiler_params=pltpu.CompilerParams(use_tc_tiling_on_sc=True,
                                             needs_layout_passes=False),
    )


def _mlp_body(xdT_ref, embT_ref, xw_ref, w1aT_ref, w1eT_ref, rT_ref, b1_ref,
              w2T_ref, b2_ref, wdT_ref, wwT_ref, boT_ref, outT_ref):
    xdT = xdT_ref[...]
    contT = xdT[_NF:, :].astype(jnp.float32)
    maskT = (xdT[:_NF, :] == 0).astype(jnp.float32)
    cnegT = -jnp.dot(w1aT_ref[...], rT_ref[...],
                     preferred_element_type=jnp.float32)
    x1 = jnp.dot(w1aT_ref[...], embT_ref[...],
                 preferred_element_type=jnp.float32)
    x1 = x1 + jnp.dot(w1eT_ref[...], contT, preferred_element_type=jnp.float32)
    x1 = x1 + jnp.dot(cnegT, maskT, preferred_element_type=jnp.float32)
    x1 = jnp.maximum(x1 + b1_ref[...], 0.0)
    x2 = jnp.dot(w2T_ref[...], x1, preferred_element_type=jnp.float32)
    x2 = jnp.maximum(x2 + b2_ref[...], 0.0)
    o = jnp.dot(wdT_ref[...], x2, preferred_element_type=jnp.float32)
    o = o + jnp.dot(wwT_ref[...], xw_ref[...].T,
                    preferred_element_type=jnp.float32)
    outT_ref[...] = o + boT_ref[...]


_BM = 8192


def _mlp_call(interpret=False):
    full = lambda a, b: pl.BlockSpec((a, b), lambda i: (0, 0))
    return pl.pallas_call(
        _mlp_body,
        grid=(_B // _BM,),
        in_specs=[
            pl.BlockSpec((_NF + _NC, _BM), lambda i: (0, i)),
            pl.BlockSpec((_DE, _BM), lambda i: (0, i)),
            pl.BlockSpec((_BM, _WD), lambda i: (i, 0)),
            full(_H1, _DE),
            full(_H1, _NC),
            full(_DE, _NF),
            full(_H1, 1),
            full(_H2, _H1),
            full(_H2, 1),
            full(_NCLS, _H2),
            full(_NCLS, _WD),
            full(_NCLS, 1),
        ],
        out_specs=pl.BlockSpec((_NCLS, _BM), lambda i: (0, i)),
        out_shape=jax.ShapeDtypeStruct((_NCLS, _B), jnp.float32),
        interpret=interpret,
    )


def kernel(X_w, X_d, tables, W1, b1, W2, b2, Wout, bout):
    t2 = jnp.transpose(tables, (0, 2, 1))
    xdT = X_d.astype(jnp.int32).T
    embT = _sc_gather()(t2, xdT)

    rows0 = tables[:, 0, :]
    R = (jnp.eye(_NF, dtype=jnp.float32)[:, :, None]
         * rows0[:, None, :]).reshape(_NF, _DE)
    outT = _mlp_call()(
        xdT, embT, X_w.astype(jnp.float32),
        W1[:_DE].T, W1[_DE:].T, R.T,
        b1.reshape(_H1, 1), W2.T, b2.reshape(_H2, 1),
        Wout[:_H2].T, Wout[_H2:].T, bout.reshape(_NCLS, 1))
    return outT.T


# speedup vs baseline: 1.0249x; 1.0249x over previous
import functools

import jax
import jax.numpy as jnp
from jax import lax
from jax.experimental import pallas as pl
from jax.experimental.pallas import tpu as pltpu
from jax.experimental.pallas import tpu_sc as plsc

_B = 16384
_NF = 26
_NC = 13
_V = 100000
_D = 16
_WD = 128
_H1 = 256
_H2 = 128
_NCLS = 2
_DE = _NF * _D

_NCORES = 2
_NSUB = 16
_NW = _NCORES * _NSUB
_PPW = _DE // _NW
_BCH = 8192
_NBCH = _B // _BCH


_UNR = 16


def _sc_plane_gather_body(t2_hbm, idxT_hbm, out_hbm, plane_v, iv_v, ob_v):
    wid = lax.axis_index("s") * _NCORES + lax.axis_index("c")

    def plane_step(p, prev_f):
        plane = wid * _PPW + p
        f = plane // _D
        d = plane % _D

        @pl.when(f != prev_f)
        def _():
            pltpu.sync_copy(idxT_hbm.at[f, :], iv_v)

        pltpu.sync_copy(t2_hbm.at[f, d, :], plane_v)

        def bchunk(c, carry2):
            def step(i, carry3):
                base = i * (16 * _UNR)
                for j in range(_UNR):
                    o = base + j * 16
                    ivv = iv_v[pl.ds(c * _BCH + o, 16)]
                    ob_v[pl.ds(o, 16)] = plsc.load_gather(plane_v, [ivv])
                return carry3

            lax.fori_loop(0, _BCH // (16 * _UNR), step, 0)
            pltpu.sync_copy(ob_v, out_hbm.at[plane, pl.ds(c * _BCH, _BCH)])
            return carry2

        lax.fori_loop(0, _NBCH, bchunk, 0)
        return f

    lax.fori_loop(0, _PPW, plane_step, -1)


@functools.cache
def _sc_gather():
    return pl.kernel(
        _sc_plane_gather_body,
        out_type=jax.ShapeDtypeStruct((_DE, _B), jnp.float32),
        mesh=plsc.VectorSubcoreMesh(core_axis_name="c", subcore_axis_name="s",
                                    num_cores=_NCORES, num_subcores=_NSUB),
        scratch_types=[
            pltpu.VMEM((_V,), jnp.float32),
            pltpu.VMEM((_B,), jnp.int32),
            pltpu.VMEM((_BCH,), jnp.float32),
        ],
        compiler_params=pltpu.CompilerParams(use_tc_tiling_on_sc=True,
                                             needs_layout_passes=False),
    )


def _mlp_body(xdT_ref, embT_ref, xw_ref, w1aT_ref, w1eT_ref, rT_ref, b1_ref,
              w2T_ref, b2_ref, wdT_ref, wwT_ref, boT_ref, outT_ref):
    xdT = xdT_ref[...]
    contT = xdT[_NF:, :].astype(jnp.float32)
    maskT = (xdT[:_NF, :] == 0).astype(jnp.float32)
    cnegT = -jnp.dot(w1aT_ref[...], rT_ref[...],
                     preferred_element_type=jnp.float32)
    x1 = jnp.dot(w1aT_ref[...], embT_ref[...],
                 preferred_element_type=jnp.float32)
    x1 = x1 + jnp.dot(w1eT_ref[...], contT, preferred_element_type=jnp.float32)
    x1 = x1 + jnp.dot(cnegT, maskT, preferred_element_type=jnp.float32)
    x1 = jnp.maximum(x1 + b1_ref[...], 0.0)
    x2 = jnp.dot(w2T_ref[...], x1, preferred_element_type=jnp.float32)
    x2 = jnp.maximum(x2 + b2_ref[...], 0.0)
    o = jnp.dot(wdT_ref[...], x2, preferred_element_type=jnp.float32)
    o = o + jnp.dot(wwT_ref[...], xw_ref[...].T,
                    preferred_element_type=jnp.float32)
    outT_ref[...] = o + boT_ref[...]


_BM = 4096


def _mlp_call(interpret=False):
    full = lambda a, b: pl.BlockSpec((a, b), lambda i: (0, 0))
    return pl.pallas_call(
        _mlp_body,
        grid=(_B // _BM,),
        in_specs=[
            pl.BlockSpec((_NF + _NC, _BM), lambda i: (0, i)),
            pl.BlockSpec((_DE, _BM), lambda i: (0, i)),
            pl.BlockSpec((_BM, _WD), lambda i: (i, 0)),
            full(_H1, _DE),
            full(_H1, _NC),
            full(_DE, _NF),
            full(_H1, 1),
            full(_H2, _H1),
            full(_H2, 1),
            full(_NCLS, _H2),
            full(_NCLS, _WD),
            full(_NCLS, 1),
        ],
        out_specs=pl.BlockSpec((_NCLS, _BM), lambda i: (0, i)),
        out_shape=jax.ShapeDtypeStruct((_NCLS, _B), jnp.float32),
        interpret=interpret,
    )


def kernel(X_w, X_d, tables, W1, b1, W2, b2, Wout, bout):
    t2 = jnp.transpose(tables, (0, 2, 1))
    xdT = X_d.astype(jnp.int32).T
    embT = _sc_gather()(t2, xdT)

    rows0 = tables[:, 0, :]
    R = (jnp.eye(_NF, dtype=jnp.float32)[:, :, None]
         * rows0[:, None, :]).reshape(_NF, _DE)
    outT = _mlp_call()(
        xdT, embT, X_w.astype(jnp.float32),
        W1[:_DE].T, W1[_DE:].T, R.T,
        b1.reshape(_H1, 1), W2.T, b2.reshape(_H2, 1),
        Wout[:_H2].T, Wout[_H2:].T, bout.reshape(_NCLS, 1))
    return outT.T
